# baseline (speedup 1.0000x reference)
import functools

import jax
import jax.numpy as jnp
from jax import lax
from jax.experimental import pallas as pl
from jax.experimental.pallas import tpu as pltpu
from jax.experimental.pallas import tpu_sc as plsc

_N = 10000
_D = 128
_L = 16
_NW = 32
_ROWS_TC = 1000


def _proj_body(x_ref, ma_ref, mb_ref, b1_ref, a_ref, b_ref):
    xv = x_ref[...]
    a_ref[...] = (
        jnp.dot(xv, ma_ref[...], preferred_element_type=jnp.float32) + b1_ref[...]
    ).astype(jnp.bfloat16)
    b_ref[...] = jnp.dot(
        xv, mb_ref[...], preferred_element_type=jnp.float32
    ).astype(jnp.bfloat16)


def _node_projections(x, ma, mb, b1row):
    grid = _N // _ROWS_TC
    return pl.pallas_call(
        _proj_body,
        grid=(grid,),
        in_specs=[
            pl.BlockSpec((_ROWS_TC, _D), lambda i: (i, 0)),
            pl.BlockSpec((_D, _D), lambda i: (0, 0)),
            pl.BlockSpec((_D, _D), lambda i: (0, 0)),
            pl.BlockSpec((1, _D), lambda i: (0, 0)),
        ],
        out_specs=[
            pl.BlockSpec((_ROWS_TC, _D), lambda i: (i, 0)),
            pl.BlockSpec((_ROWS_TC, _D), lambda i: (i, 0)),
        ],
        out_shape=[
            jax.ShapeDtypeStruct((_N, _D), jnp.bfloat16),
            jax.ShapeDtypeStruct((_N, _D), jnp.bfloat16),
        ],
    )(x, ma, mb, b1row)


def _sc_edge_kernel(num_edges, chunk):
    nchunk_total = num_edges // chunk
    assert nchunk_total % _NW == 0
    nchunk = nchunk_total // _NW
    epw = nchunk * chunk

    mesh = plsc.VectorSubcoreMesh(core_axis_name="c", subcore_axis_name="s")

    @functools.partial(
        pl.kernel,
        mesh=mesh,
        compiler_params=pltpu.CompilerParams(
            needs_layout_passes=False, use_tc_tiling_on_sc=False),
        out_type=jax.ShapeDtypeStruct((num_edges,), jnp.float32),
        scratch_types=[
            pltpu.VMEM((epw,), jnp.int32),
            pltpu.VMEM((epw,), jnp.int32),
            pltpu.VMEM((4, chunk, _D // 2), jnp.int32),
            pltpu.VMEM((4, chunk, _D // 2), jnp.int32),
            pltpu.VMEM((epw,), jnp.float32),
            pltpu.VMEM((chunk // _L, _L, _L), jnp.float32),
            pltpu.VMEM((_D // 2,), jnp.int32),
            pltpu.VMEM((_L,), jnp.float32),
            pltpu.SemaphoreType.DMA,
            pltpu.SemaphoreType.DMA,
            pltpu.SemaphoreType.DMA,
            pltpu.SemaphoreType.DMA,
            pltpu.SemaphoreType.DMA,
            pltpu.SemaphoreType.DMA,
            pltpu.SemaphoreType.DMA,
            pltpu.SemaphoreType.DMA,
        ],
    )
    def k(a_hbm, b_hbm, src_hbm, tar_hbm, w2_hbm, b2_hbm, out_hbm,
          idxs_v, idxt_v, rows_a, rows_b, out_v, tr_v, w2_v, b2_v,
          sem_a0, sem_a1, sem_a2, sem_a3, sem_b0, sem_b1, sem_b2, sem_b3):
        wid = lax.axis_index("s") * 2 + lax.axis_index("c")
        base0 = wid * epw
        pltpu.sync_copy(w2_hbm, w2_v)
        pltpu.sync_copy(b2_hbm, b2_v)
        pltpu.sync_copy(src_hbm.at[pl.ds(base0, epw)], idxs_v)
        pltpu.sync_copy(tar_hbm.at[pl.ds(base0, epw)], idxt_v)
        b2vec = b2_v[...]
        w2s = []
        for j in range(_D // (2 * _L)):
            w32 = plsc.bitcast(w2_v[pl.ds(j * _L, _L)], jnp.bfloat16)
            we, wo = plsc.unpack(w32, format=plsc.PackFormat.INTERLEAVED)
            w2s += [we, wo]
        iota16 = lax.iota(jnp.int32, _L)
        zero16 = jnp.zeros((_L,), jnp.float32)
        sems = [(sem_a0, sem_b0), (sem_a1, sem_b1),
                (sem_a2, sem_b2), (sem_a3, sem_b3)]
        nbuf = 4

        def gather_descs(g, b):
            sa, sb = sems[b]
            idx_a = idxs_v.at[pl.ds(g * chunk, chunk)]
            idx_b = idxt_v.at[pl.ds(g * chunk, chunk)]
            return (pltpu.make_async_copy(a_hbm.at[idx_a], rows_a.at[b], sa),
                    pltpu.make_async_copy(b_hbm.at[idx_b], rows_b.at[b], sb))

        def start_gather(g, b):
            for cp in gather_descs(g, b):
                cp.start()

        def compute_chunk(g, b):
            for cp in gather_descs(g, b):
                cp.wait()

            @plsc.parallel_loop(0, chunk // _L)
            def grp_body(t):
                e0 = t * _L
                tfull = jnp.full((_L,), t, jnp.int32)
                accs = []
                for i in range(_L):
                    e = e0 + i
                    acc_e = None
                    acc_o = None
                    for j in range(_D // (2 * _L)):
                        va = plsc.bitcast(
                            rows_a[b, e, pl.ds(j * _L, _L)], jnp.bfloat16)
                        vb = plsc.bitcast(
                            rows_b[b, e, pl.ds(j * _L, _L)], jnp.bfloat16)
                        h = jnp.maximum(va + vb, jnp.bfloat16(0.0))
                        he, ho = plsc.unpack(
                            h, format=plsc.PackFormat.INTERLEAVED)
                        if acc_e is None:
                            acc_e = he * w2s[0]
                            acc_o = ho * w2s[1]
                        else:
                            acc_e = acc_e + he * w2s[2 * j]
                            acc_o = acc_o + ho * w2s[2 * j + 1]
                    accs.append(acc_e + acc_o)
                for i in range(_L):
                    tr_v[t, i, :] = accs[i]
                outv = b2vec
                for kk in range(_L):
                    colk = plsc.load_gather(
                        tr_v, [tfull, iota16, jnp.full((_L,), kk, jnp.int32)])
                    outv = outv + colk
                out_v[pl.ds(g * chunk + e0, _L)] = outv

        for b in range(nbuf):
            start_gather(b, b)

        def ring_body(h, carry):
            for b in range(nbuf):
                g = h * nbuf + b
                compute_chunk(g, b)

                @pl.when(g + nbuf < nchunk)
                def _():
                    start_gather(g + nbuf, b)
            return carry

        lax.fori_loop(0, nchunk // nbuf, ring_body, 0)
        for r in range(nchunk % nbuf):
            g = (nchunk // nbuf) * nbuf + r
            compute_chunk(g, g % nbuf)
        pltpu.sync_copy(out_v, out_hbm.at[pl.ds(base0, epw)])

    return k


def kernel(x, pos_edge_index, neg_edge_index, W1, b1, W2, b2):
    num_edges = 2 * pos_edge_index.shape[1]
    fwd = jnp.concatenate([pos_edge_index, neg_edge_index], axis=-1)
    src = fwd[0]
    tar = fwd[1]

    ma = W1[:, :_D].T
    mb = W1[:, _D:].T
    b1row = b1.reshape(1, _D)
    a_tab, b_tab = _node_projections(x, ma, mb, b1row)

    a_tab32 = jax.lax.bitcast_convert_type(
        a_tab.reshape(_N, _D // 2, 2), jnp.int32)
    b_tab32 = jax.lax.bitcast_convert_type(
        b_tab.reshape(_N, _D // 2, 2), jnp.int32)
    w2 = jax.lax.bitcast_convert_type(
        W2.reshape(_D).astype(jnp.bfloat16).reshape(_D // 2, 2), jnp.int32)
    b2v = jnp.full((_L,), b2[0], jnp.float32)

    out = _sc_edge_kernel(num_edges, chunk=80)(
        a_tab32, b_tab32, src, tar, w2, b2v)
    return out.reshape(num_edges, 1)


# speedup vs baseline: 1.5713x; 1.5713x over previous
import functools

import jax
import jax.numpy as jnp
from jax import lax
from jax.experimental import pallas as pl
from jax.experimental.pallas import tpu as pltpu
from jax.experimental.pallas import tpu_sc as plsc

_N = 10000
_D = 128
_L = 16
_NW = 32
_ROWS_TC = 1000


def _proj_body(x_ref, ma_ref, mb_ref, b1_ref, a_ref, b_ref):
    xv = x_ref[...]
    a_ref[...] = (
        jnp.dot(xv, ma_ref[...], preferred_element_type=jnp.float32) + b1_ref[...]
    ).astype(jnp.bfloat16)
    b_ref[...] = jnp.dot(
        xv, mb_ref[...], preferred_element_type=jnp.float32
    ).astype(jnp.bfloat16)


def _node_projections(x, ma, mb, b1row):
    grid = _N // _ROWS_TC
    return pl.pallas_call(
        _proj_body,
        grid=(grid,),
        in_specs=[
            pl.BlockSpec((_ROWS_TC, _D), lambda i: (i, 0)),
            pl.BlockSpec((_D, _D), lambda i: (0, 0)),
            pl.BlockSpec((_D, _D), lambda i: (0, 0)),
            pl.BlockSpec((1, _D), lambda i: (0, 0)),
        ],
        out_specs=[
            pl.BlockSpec((_ROWS_TC, _D), lambda i: (i, 0)),
            pl.BlockSpec((_ROWS_TC, _D), lambda i: (i, 0)),
        ],
        out_shape=[
            jax.ShapeDtypeStruct((_N, _D), jnp.bfloat16),
            jax.ShapeDtypeStruct((_N, _D), jnp.bfloat16),
        ],
    )(x, ma, mb, b1row)


def _sc_edge_kernel(num_edges, chunk):
    nchunk_total = num_edges // chunk
    assert nchunk_total % _NW == 0
    nchunk = nchunk_total // _NW
    epw = nchunk * chunk

    mesh = plsc.VectorSubcoreMesh(core_axis_name="c", subcore_axis_name="s")

    @functools.partial(
        pl.kernel,
        mesh=mesh,
        compiler_params=pltpu.CompilerParams(
            needs_layout_passes=False, use_tc_tiling_on_sc=False),
        out_type=jax.ShapeDtypeStruct((num_edges,), jnp.float32),
        scratch_types=[
            pltpu.VMEM((epw,), jnp.int32),
            pltpu.VMEM((epw,), jnp.int32),
            pltpu.VMEM((4, chunk, _D // 2), jnp.int32),
            pltpu.VMEM((4, chunk, _D // 2), jnp.int32),
            pltpu.VMEM((epw,), jnp.float32),
            pltpu.VMEM((chunk // _L, _L, _L), jnp.float32),
            pltpu.VMEM((_D // 2,), jnp.int32),
            pltpu.VMEM((_L,), jnp.float32),
            pltpu.SemaphoreType.DMA,
            pltpu.SemaphoreType.DMA,
            pltpu.SemaphoreType.DMA,
            pltpu.SemaphoreType.DMA,
            pltpu.SemaphoreType.DMA,
            pltpu.SemaphoreType.DMA,
            pltpu.SemaphoreType.DMA,
            pltpu.SemaphoreType.DMA,
        ],
    )
    def k(a_hbm, b_hbm, src_hbm, tar_hbm, w2_hbm, b2_hbm, out_hbm,
          idxs_v, idxt_v, rows_a, rows_b, out_v, tr_v, w2_v, b2_v,
          sem_a0, sem_a1, sem_a2, sem_a3, sem_b0, sem_b1, sem_b2, sem_b3):
        wid = lax.axis_index("s") * 2 + lax.axis_index("c")
        base0 = wid * epw
        pltpu.sync_copy(w2_hbm, w2_v)
        pltpu.sync_copy(b2_hbm, b2_v)
        pltpu.sync_copy(src_hbm.at[pl.ds(base0, epw)], idxs_v)
        pltpu.sync_copy(tar_hbm.at[pl.ds(base0, epw)], idxt_v)
        b2vec = b2_v[...]
        w2s = []
        for j in range(_D // (2 * _L)):
            w32 = plsc.bitcast(w2_v[pl.ds(j * _L, _L)], jnp.bfloat16)
            we, wo = plsc.unpack(w32, format=plsc.PackFormat.INTERLEAVED)
            w2s += [we, wo]
        iota16 = lax.iota(jnp.int32, _L)
        zero16 = jnp.zeros((_L,), jnp.float32)
        sems = [(sem_a0, sem_b0), (sem_a1, sem_b1),
                (sem_a2, sem_b2), (sem_a3, sem_b3)]
        nbuf = 4

        def gather_descs(g, b):
            sa, sb = sems[b]
            idx_a = idxs_v.at[pl.ds(g * chunk, chunk)]
            idx_b = idxt_v.at[pl.ds(g * chunk, chunk)]
            return (pltpu.make_async_copy(a_hbm.at[idx_a], rows_a.at[b], sa),
                    pltpu.make_async_copy(b_hbm.at[idx_b], rows_b.at[b], sb))

        def start_gather(g, b):
            for cp in gather_descs(g, b):
                cp.start()

        def compute_chunk(g, b):
            for cp in gather_descs(g, b):
                cp.wait()

            @plsc.parallel_loop(0, chunk // _L)
            def grp_body(t):
                e0 = t * _L
                tfull = jnp.full((_L,), t, jnp.int32)
                for i in range(_L):
                    e = e0 + i
                    acc_e = None
                    acc_o = None
                    for j in range(_D // (2 * _L)):
                        va = plsc.bitcast(
                            rows_a[b, e, pl.ds(j * _L, _L)], jnp.bfloat16)
                        vb = plsc.bitcast(
                            rows_b[b, e, pl.ds(j * _L, _L)], jnp.bfloat16)
                        h = jnp.maximum(va + vb, jnp.bfloat16(0.0))
                        he, ho = plsc.unpack(
                            h, format=plsc.PackFormat.INTERLEAVED)
                        if acc_e is None:
                            acc_e = he * w2s[0]
                            acc_o = ho * w2s[1]
                        else:
                            acc_e = acc_e + he * w2s[2 * j]
                            acc_o = acc_o + ho * w2s[2 * j + 1]
                    tr_v[t, i, :] = acc_e + acc_o
                outv = b2vec
                for kk in range(_L):
                    colk = plsc.load_gather(
                        tr_v, [tfull, iota16, jnp.full((_L,), kk, jnp.int32)])
                    outv = outv + colk
                out_v[pl.ds(g * chunk + e0, _L)] = outv

        for b in range(nbuf):
            start_gather(b, b)

        def ring_body(h, carry):
            for b in range(nbuf):
                g = h * nbuf + b
                compute_chunk(g, b)

                @pl.when(g + nbuf < nchunk)
                def _():
                    start_gather(g + nbuf, b)
            return carry

        lax.fori_loop(0, nchunk // nbuf, ring_body, 0)
        for r in range(nchunk % nbuf):
            g = (nchunk // nbuf) * nbuf + r
            compute_chunk(g, g % nbuf)
        pltpu.sync_copy(out_v, out_hbm.at[pl.ds(base0, epw)])

    return k


def kernel(x, pos_edge_index, neg_edge_index, W1, b1, W2, b2):
    num_edges = 2 * pos_edge_index.shape[1]
    fwd = jnp.concatenate([pos_edge_index, neg_edge_index], axis=-1)
    src = fwd[0]
    tar = fwd[1]

    ma = W1[:, :_D].T
    mb = W1[:, _D:].T
    b1row = b1.reshape(1, _D)
    a_tab, b_tab = _node_projections(x, ma, mb, b1row)

    a_tab32 = jax.lax.bitcast_convert_type(
        a_tab.reshape(_N, _D // 2, 2), jnp.int32)
    b_tab32 = jax.lax.bitcast_convert_type(
        b_tab.reshape(_N, _D // 2, 2), jnp.int32)
    w2 = jax.lax.bitcast_convert_type(
        W2.reshape(_D).astype(jnp.bfloat16).reshape(_D // 2, 2), jnp.int32)
    b2v = jnp.full((_L,), b2[0], jnp.float32)

    out = _sc_edge_kernel(num_edges, chunk=80)(
        a_tab32, b_tab32, src, tar, w2, b2v)
    return out.reshape(num_edges, 1)


# speedup vs baseline: 1.9698x; 1.2536x over previous
import functools

import jax
import jax.numpy as jnp
from jax import lax
from jax.experimental import pallas as pl
from jax.experimental.pallas import tpu as pltpu
from jax.experimental.pallas import tpu_sc as plsc

_N = 10000
_D = 128
_L = 16
_NW = 32
_ROWS_TC = 1000


def _proj_body(x_ref, ma_ref, mb_ref, b1_ref, a_ref, b_ref):
    xv = x_ref[...]
    a_ref[...] = (
        jnp.dot(xv, ma_ref[...], preferred_element_type=jnp.float32) + b1_ref[...]
    ).astype(jnp.bfloat16)
    b_ref[...] = jnp.dot(
        xv, mb_ref[...], preferred_element_type=jnp.float32
    ).astype(jnp.bfloat16)


def _node_projections(x, ma, mb, b1row):
    grid = _N // _ROWS_TC
    return pl.pallas_call(
        _proj_body,
        grid=(grid,),
        in_specs=[
            pl.BlockSpec((_ROWS_TC, _D), lambda i: (i, 0)),
            pl.BlockSpec((_D, _D), lambda i: (0, 0)),
            pl.BlockSpec((_D, _D), lambda i: (0, 0)),
            pl.BlockSpec((1, _D), lambda i: (0, 0)),
        ],
        out_specs=[
            pl.BlockSpec((_ROWS_TC, _D), lambda i: (i, 0)),
            pl.BlockSpec((_ROWS_TC, _D), lambda i: (i, 0)),
        ],
        out_shape=[
            jax.ShapeDtypeStruct((_N, _D), jnp.bfloat16),
            jax.ShapeDtypeStruct((_N, _D), jnp.bfloat16),
        ],
    )(x, ma, mb, b1row)


def _sc_edge_kernel(num_edges, chunk):
    nchunk_total = num_edges // chunk
    assert nchunk_total % _NW == 0
    nchunk = nchunk_total // _NW
    epw = nchunk * chunk

    mesh = plsc.VectorSubcoreMesh(core_axis_name="c", subcore_axis_name="s")

    @functools.partial(
        pl.kernel,
        mesh=mesh,
        compiler_params=pltpu.CompilerParams(
            needs_layout_passes=False, use_tc_tiling_on_sc=False),
        out_type=jax.ShapeDtypeStruct((num_edges,), jnp.float32),
        scratch_types=[
            pltpu.VMEM((epw,), jnp.int32),
            pltpu.VMEM((epw,), jnp.int32),
            pltpu.VMEM((4, chunk, _D // 2), jnp.int32),
            pltpu.VMEM((4, chunk, _D // 2), jnp.int32),
            pltpu.VMEM((epw,), jnp.float32),
            pltpu.VMEM((chunk // _L, _L, _L), jnp.float32),
            pltpu.VMEM((_D // 2,), jnp.int32),
            pltpu.VMEM((_L,), jnp.float32),
            pltpu.SemaphoreType.DMA,
            pltpu.SemaphoreType.DMA,
            pltpu.SemaphoreType.DMA,
            pltpu.SemaphoreType.DMA,
            pltpu.SemaphoreType.DMA,
            pltpu.SemaphoreType.DMA,
            pltpu.SemaphoreType.DMA,
            pltpu.SemaphoreType.DMA,
        ],
    )
    def k(a_hbm, b_hbm, src_hbm, tar_hbm, w2_hbm, b2_hbm, out_hbm,
          idxs_v, idxt_v, rows_a, rows_b, out_v, tr_v, w2_v, b2_v,
          sem_a0, sem_a1, sem_a2, sem_a3, sem_b0, sem_b1, sem_b2, sem_b3):
        wid = lax.axis_index("s") * 2 + lax.axis_index("c")
        base0 = wid * epw
        pltpu.sync_copy(w2_hbm, w2_v)
        pltpu.sync_copy(b2_hbm, b2_v)
        pltpu.sync_copy(src_hbm.at[pl.ds(base0, epw)], idxs_v)
        pltpu.sync_copy(tar_hbm.at[pl.ds(base0, epw)], idxt_v)
        b2vec = b2_v[...]
        w2s = [plsc.bitcast(w2_v[pl.ds(j * _L, _L)], jnp.bfloat16)
               for j in range(_D // (2 * _L))]
        iota16 = lax.iota(jnp.int32, _L)
        sems = [(sem_a0, sem_b0), (sem_a1, sem_b1),
                (sem_a2, sem_b2), (sem_a3, sem_b3)]
        nbuf = 2

        def gather_descs(g, b):
            sa, sb = sems[b]
            idx_a = idxs_v.at[pl.ds(g * chunk, chunk)]
            idx_b = idxt_v.at[pl.ds(g * chunk, chunk)]
            return (pltpu.make_async_copy(a_hbm.at[idx_a], rows_a.at[b], sa),
                    pltpu.make_async_copy(b_hbm.at[idx_b], rows_b.at[b], sb))

        def start_gather(g, b):
            for cp in gather_descs(g, b):
                cp.start()

        def compute_chunk(g, b):
            for cp in gather_descs(g, b):
                cp.wait()

            def grp_body(t, c):
                e0 = t * _L
                tfull = jnp.full((_L,), t, jnp.int32)
                for i in range(_L):
                    e = e0 + i
                    acc = None
                    for j in range(_D // (2 * _L)):
                        va = plsc.bitcast(
                            rows_a[b, e, pl.ds(j * _L, _L)], jnp.bfloat16)
                        vb = plsc.bitcast(
                            rows_b[b, e, pl.ds(j * _L, _L)], jnp.bfloat16)
                        h = jnp.maximum(va + vb, jnp.bfloat16(0.0))
                        p = h * w2s[j]
                        acc = p if acc is None else acc + p
                    ae, ao = plsc.unpack(acc, format=plsc.PackFormat.INTERLEAVED)
                    tr_v[t, i, :] = ae + ao
                outv = b2vec
                for kk in range(_L):
                    colk = plsc.load_gather(
                        tr_v, [tfull, iota16, jnp.full((_L,), kk, jnp.int32)])
                    outv = outv + colk
                out_v[pl.ds(g * chunk + e0, _L)] = outv
                return c

            lax.fori_loop(0, chunk // _L, grp_body, 0)

        for b in range(nbuf):
            start_gather(b, b)

        def ring_body(h, carry):
            for b in range(nbuf):
                g = h * nbuf + b
                compute_chunk(g, b)

                @pl.when(g + nbuf < nchunk)
                def _():
                    start_gather(g + nbuf, b)
            return carry

        lax.fori_loop(0, nchunk // nbuf, ring_body, 0)
        for r in range(nchunk % nbuf):
            g = (nchunk // nbuf) * nbuf + r
            compute_chunk(g, g % nbuf)
        pltpu.sync_copy(out_v, out_hbm.at[pl.ds(base0, epw)])

    return k


def kernel(x, pos_edge_index, neg_edge_index, W1, b1, W2, b2):
    num_edges = 2 * pos_edge_index.shape[1]
    fwd = jnp.concatenate([pos_edge_index, neg_edge_index], axis=-1)
    src = fwd[0]
    tar = fwd[1]

    ma = W1[:, :_D].T
    mb = W1[:, _D:].T
    b1row = b1.reshape(1, _D)
    a_tab, b_tab = _node_projections(x, ma, mb, b1row)

    a_tab32 = jax.lax.bitcast_convert_type(
        a_tab.reshape(_N, _D // 2, 2), jnp.int32)
    b_tab32 = jax.lax.bitcast_convert_type(
        b_tab.reshape(_N, _D // 2, 2), jnp.int32)
    w2 = jax.lax.bitcast_convert_type(
        W2.reshape(_D).astype(jnp.bfloat16).reshape(_D // 2, 2), jnp.int32)
    b2v = jnp.full((_L,), b2[0], jnp.float32)

    out = _sc_edge_kernel(num_edges, chunk=80)(
        a_tab32, b_tab32, src, tar, w2, b2v)
    return out.reshape(num_edges, 1)


# speedup vs baseline: 2.6417x; 1.3411x over previous
import functools

import jax
import jax.numpy as jnp
from jax import lax
from jax.experimental import pallas as pl
from jax.experimental.pallas import tpu as pltpu
from jax.experimental.pallas import tpu_sc as plsc

_N = 10000
_D = 128
_L = 16
_NW = 32
_ROWS_TC = 1000


def _proj_body(x_ref, ma_ref, mb_ref, b1_ref, a_ref, b_ref):
    xv = x_ref[...]
    a_ref[...] = (
        jnp.dot(xv, ma_ref[...], preferred_element_type=jnp.float32) + b1_ref[...]
    ).astype(jnp.bfloat16)
    b_ref[...] = jnp.dot(
        xv, mb_ref[...], preferred_element_type=jnp.float32
    ).astype(jnp.bfloat16)


def _node_projections(x, ma, mb, b1row):
    grid = _N // _ROWS_TC
    return pl.pallas_call(
        _proj_body,
        grid=(grid,),
        in_specs=[
            pl.BlockSpec((_ROWS_TC, _D), lambda i: (i, 0)),
            pl.BlockSpec((_D, _D), lambda i: (0, 0)),
            pl.BlockSpec((_D, _D), lambda i: (0, 0)),
            pl.BlockSpec((1, _D), lambda i: (0, 0)),
        ],
        out_specs=[
            pl.BlockSpec((_ROWS_TC, _D), lambda i: (i, 0)),
            pl.BlockSpec((_ROWS_TC, _D), lambda i: (i, 0)),
        ],
        out_shape=[
            jax.ShapeDtypeStruct((_N, _D), jnp.bfloat16),
            jax.ShapeDtypeStruct((_N, _D), jnp.bfloat16),
        ],
    )(x, ma, mb, b1row)


def _sc_edge_kernel(num_edges, chunk):
    nchunk_total = num_edges // chunk
    assert nchunk_total % _NW == 0
    nchunk = nchunk_total // _NW
    epw = nchunk * chunk

    mesh = plsc.VectorSubcoreMesh(core_axis_name="c", subcore_axis_name="s")

    @functools.partial(
        pl.kernel,
        mesh=mesh,
        compiler_params=pltpu.CompilerParams(
            needs_layout_passes=False, use_tc_tiling_on_sc=False),
        out_type=jax.ShapeDtypeStruct((num_edges,), jnp.float32),
        scratch_types=[
            pltpu.VMEM((epw,), jnp.int32),
            pltpu.VMEM((epw,), jnp.int32),
            pltpu.VMEM((4, chunk, _D // 2), jnp.int32),
            pltpu.VMEM((4, chunk, _D // 2), jnp.int32),
            pltpu.VMEM((epw,), jnp.float32),
            pltpu.VMEM((chunk // _L, _L, _L), jnp.float32),
            pltpu.VMEM((_D // 2,), jnp.int32),
            pltpu.VMEM((_L,), jnp.float32),
            pltpu.SemaphoreType.DMA,
            pltpu.SemaphoreType.DMA,
            pltpu.SemaphoreType.DMA,
            pltpu.SemaphoreType.DMA,
            pltpu.SemaphoreType.DMA,
            pltpu.SemaphoreType.DMA,
            pltpu.SemaphoreType.DMA,
            pltpu.SemaphoreType.DMA,
        ],
    )
    def k(a_hbm, b_hbm, src_hbm, tar_hbm, w2_hbm, b2_hbm, out_hbm,
          idxs_v, idxt_v, rows_a, rows_b, out_v, tr_v, w2_v, b2_v,
          sem_a0, sem_a1, sem_a2, sem_a3, sem_b0, sem_b1, sem_b2, sem_b3):
        wid = lax.axis_index("s") * 2 + lax.axis_index("c")
        base0 = wid * epw
        pltpu.sync_copy(w2_hbm, w2_v)
        pltpu.sync_copy(b2_hbm, b2_v)
        pltpu.sync_copy(src_hbm.at[pl.ds(base0, epw)], idxs_v)
        pltpu.sync_copy(tar_hbm.at[pl.ds(base0, epw)], idxt_v)
        b2vec = b2_v[...]
        w2s = [plsc.bitcast(w2_v[pl.ds(j * _L, _L)], jnp.bfloat16)
               for j in range(_D // (2 * _L))]
        iota16 = lax.iota(jnp.int32, _L)
        sems = [(sem_a0, sem_b0), (sem_a1, sem_b1),
                (sem_a2, sem_b2), (sem_a3, sem_b3)]
        nbuf = 3

        def gather_descs(g, b):
            sa, sb = sems[b]
            idx_a = idxs_v.at[pl.ds(g * chunk, chunk)]
            idx_b = idxt_v.at[pl.ds(g * chunk, chunk)]
            return (pltpu.make_async_copy(a_hbm.at[idx_a], rows_a.at[b], sa),
                    pltpu.make_async_copy(b_hbm.at[idx_b], rows_b.at[b], sb))

        def start_gather(g, b):
            for cp in gather_descs(g, b):
                cp.start()

        def compute_chunk(g, b):
            for cp in gather_descs(g, b):
                cp.wait()

            def grp_body(t, c):
                e0 = t * _L
                tfull = jnp.full((_L,), t, jnp.int32)
                nsub = 8
                for sg in range(_L // nsub):
                    accs = []
                    for i in range(nsub):
                        e = e0 + sg * nsub + i
                        acc0 = None
                        acc1 = None
                        for j in range(_D // (2 * _L)):
                            va = plsc.bitcast(
                                rows_a[b, e, pl.ds(j * _L, _L)], jnp.bfloat16)
                            vb = plsc.bitcast(
                                rows_b[b, e, pl.ds(j * _L, _L)], jnp.bfloat16)
                            h = jnp.maximum(va + vb, jnp.bfloat16(0.0))
                            p = h * w2s[j]
                            if j % 2 == 0:
                                acc0 = p if acc0 is None else acc0 + p
                            else:
                                acc1 = p if acc1 is None else acc1 + p
                        ae, ao = plsc.unpack(
                            acc0 + acc1, format=plsc.PackFormat.INTERLEAVED)
                        accs.append(ae + ao)
                    for i in range(nsub):
                        tr_v[t, sg * nsub + i, :] = accs[i]
                outv = b2vec
                for kk in range(_L):
                    colk = plsc.load_gather(
                        tr_v, [tfull, iota16, jnp.full((_L,), kk, jnp.int32)])
                    outv = outv + colk
                out_v[pl.ds(g * chunk + e0, _L)] = outv
                return c

            lax.fori_loop(0, chunk // _L, grp_body, 0)

        for b in range(nbuf):
            start_gather(b, b)

        def ring_body(h, carry):
            for b in range(nbuf):
                g = h * nbuf + b
                compute_chunk(g, b)

                @pl.when(g + nbuf < nchunk)
                def _():
                    start_gather(g + nbuf, b)
            return carry

        lax.fori_loop(0, nchunk // nbuf, ring_body, 0)
        for r in range(nchunk % nbuf):
            g = (nchunk // nbuf) * nbuf + r
            compute_chunk(g, g % nbuf)
        pltpu.sync_copy(out_v, out_hbm.at[pl.ds(base0, epw)])

    return k


def kernel(x, pos_edge_index, neg_edge_index, W1, b1, W2, b2):
    num_edges = 2 * pos_edge_index.shape[1]
    fwd = jnp.concatenate([pos_edge_index, neg_edge_index], axis=-1)
    src = fwd[0]
    tar = fwd[1]

    ma = W1[:, :_D].T
    mb = W1[:, _D:].T
    b1row = b1.reshape(1, _D)
    a_tab, b_tab = _node_projections(x, ma, mb, b1row)

    a_tab32 = jax.lax.bitcast_convert_type(
        a_tab.reshape(_N, _D // 2, 2), jnp.int32)
    b_tab32 = jax.lax.bitcast_convert_type(
        b_tab.reshape(_N, _D // 2, 2), jnp.int32)
    w2 = jax.lax.bitcast_convert_type(
        W2.reshape(_D).astype(jnp.bfloat16).reshape(_D // 2, 2), jnp.int32)
    b2v = jnp.full((_L,), b2[0], jnp.float32)

    out = _sc_edge_kernel(num_edges, chunk=80)(
        a_tab32, b_tab32, src, tar, w2, b2v)
    return out.reshape(num_edges, 1)
